# baseline (speedup 1.0000x reference)
import functools

import jax
import jax.numpy as jnp
from jax import lax
from jax.experimental import pallas as pl
from jax.experimental.pallas import tpu as pltpu
from jax.experimental.pallas import tpu_sc as plsc


def _route_body(C, E, x_ref, wg_ref, bg_ref, slot_ref, scale_ref, counts_ref):
  i = pl.program_id(0)

  @pl.when(i == 0)
  def _():
    counts_ref[...] = jnp.zeros_like(counts_ref)

  logits = jnp.dot(x_ref[...], wg_ref[...],
                   preferred_element_type=jnp.float32) + bg_ref[...]
  m = jnp.max(logits, axis=1, keepdims=True)
  gate = 1.0 / jnp.sum(jnp.exp(logits - m), axis=1, keepdims=True)
  e_iota = lax.broadcasted_iota(jnp.int32, logits.shape, 1)
  idx = jnp.min(jnp.where(logits == m, e_iota, E), axis=1,
                keepdims=True)
  onehot = (e_iota == idx).astype(jnp.float32)

  T = onehot.shape[0]
  r = lax.broadcasted_iota(jnp.int32, (T, T), 0)
  c = lax.broadcasted_iota(jnp.int32, (T, T), 1)
  tri = (c < r).astype(jnp.float32)
  loc = jnp.dot(tri, onehot, preferred_element_type=jnp.float32)
  loc = loc + counts_ref[...]
  counts_ref[...] = counts_ref[...] + jnp.sum(onehot, axis=0, keepdims=True)
  loc_i = jnp.sum(loc * onehot, axis=1, keepdims=True).astype(jnp.int32)

  valid = loc_i < C
  slot = idx * C + loc_i
  slot_ref[...] = jnp.where(valid, slot, E * C)
  scale_ref[...] = gate


def _route(xr, Wg, bg, C, E, block_t):
  S, M = xr.shape
  n = S // block_t
  return pl.pallas_call(
      functools.partial(_route_body, C, E),
      grid=(n,),
      in_specs=[
          pl.BlockSpec((block_t, M), lambda i: (i, 0)),
          pl.BlockSpec((M, E), lambda i: (0, 0)),
          pl.BlockSpec((1, E), lambda i: (0, 0)),
      ],
      out_specs=[
          pl.BlockSpec((block_t, 1), lambda i: (i, 0)),
          pl.BlockSpec((block_t, 1), lambda i: (i, 0)),
      ],
      out_shape=[
          jax.ShapeDtypeStruct((S, 1), jnp.int32),
          jax.ShapeDtypeStruct((S, 1), jnp.float32),
      ],
      scratch_shapes=[pltpu.VMEM((1, E), jnp.float32)],
  )(xr, Wg, bg.reshape(1, E))


def _ffn_body(E, x_ref, w1_ref, b1_ref, w2_ref, b2_ref, g_ref, out_ref):
  e = pl.program_id(0)
  f = pl.program_id(1)
  nf = pl.num_programs(1)

  @pl.when(e == E)
  def _():
    out_ref[...] = jnp.zeros_like(out_ref)

  @pl.when(e < E)
  def _():
    fb = w1_ref.shape[2]
    h = jnp.dot(x_ref[...], w1_ref[0], preferred_element_type=jnp.float32)
    b1blk = b1_ref[0, :, pl.ds(f * fb, fb)]
    h = jnp.maximum(h + b1blk, 0.0)
    contrib = jnp.dot(h, w2_ref[0], preferred_element_type=jnp.float32)

    @pl.when(f == 0)
    def _():
      out_ref[...] = contrib

    @pl.when(f != 0)
    def _():
      out_ref[...] += contrib

    @pl.when(f == nf - 1)
    def _():
      out_ref[...] = (out_ref[...] + b2_ref[0]) * g_ref[...]


def _ffn(disp, W1, b1, W2, b2, gfs, C):
  E, M, F = W1.shape
  nf = 2
  fb = F // nf
  ec = lambda e: jnp.minimum(e, E - 1)
  return pl.pallas_call(
      functools.partial(_ffn_body, E),
      grid=(E + 1, nf),
      in_specs=[
          pl.BlockSpec((C, M), lambda e, f: (ec(e), 0)),
          pl.BlockSpec((1, M, fb), lambda e, f: (ec(e), 0, f)),
          pl.BlockSpec((1, 1, F), lambda e, f: (ec(e), 0, 0)),
          pl.BlockSpec((1, fb, M), lambda e, f: (ec(e), f, 0)),
          pl.BlockSpec((1, 1, M), lambda e, f: (ec(e), 0, 0)),
          pl.BlockSpec((C, 1), lambda e, f: (ec(e), 0)),
      ],
      out_specs=pl.BlockSpec((C, M), lambda e, f: (e, 0)),
      out_shape=jax.ShapeDtypeStruct(((E + 1) * C, M), jnp.float32),
  )(disp, W1, b1.reshape(E, 1, F), W2, b2.reshape(E, 1, M), gfs)


_NC = 2
_NS = 16
_NW = _NC * _NS


def _make_dispatch(S, M, n_rows, K, CH):
  mesh = plsc.VectorSubcoreMesh(core_axis_name="c", subcore_axis_name="s")

  @functools.partial(
      pl.kernel,
      mesh=mesh,
      out_type=(
          jax.ShapeDtypeStruct((n_rows, M), jnp.float32),
          jax.ShapeDtypeStruct((n_rows,), jnp.float32),
      ),
      scratch_types=[
          pltpu.VMEM((K, CH), jnp.int32),
          pltpu.VMEM((CH, M), jnp.float32),
          pltpu.VMEM((K, CH), jnp.float32),
          pltpu.SemaphoreType.DMA,
      ],
  )
  def dispatch(x_hbm, slot_hbm, scale_hbm, out_hbm, gfs_hbm, idx_v, rows_v,
               scale_v, sem):
    wid = lax.axis_index("s") * _NC + lax.axis_index("c")
    pltpu.sync_copy(slot_hbm.at[wid], idx_v)
    pltpu.sync_copy(scale_hbm.at[wid], scale_v)
    for j in range(K):
      base = wid * (K * CH) + j * CH
      pltpu.sync_copy(x_hbm.at[pl.ds(base, CH)], rows_v)
      pltpu.async_copy(rows_v, out_hbm.at[idx_v.at[j]], sem).wait()
      pltpu.async_copy(scale_v.at[j], gfs_hbm.at[idx_v.at[j]], sem).wait()

  return dispatch


def _make_decode(S, M, K, CH):
  mesh = plsc.VectorSubcoreMesh(core_axis_name="c", subcore_axis_name="s")

  @functools.partial(
      pl.kernel,
      mesh=mesh,
      out_type=jax.ShapeDtypeStruct((S, M), jnp.float32),
      scratch_types=[
          pltpu.VMEM((K, CH), jnp.int32),
          pltpu.VMEM((CH, M), jnp.float32),
          pltpu.SemaphoreType.DMA,
      ],
  )
  def decode(eo_hbm, slot_hbm, out_hbm, idx_v, rows_v, sem):
    wid = lax.axis_index("s") * _NC + lax.axis_index("c")
    pltpu.sync_copy(slot_hbm.at[wid], idx_v)
    for j in range(K):
      base = wid * (K * CH) + j * CH
      pltpu.async_copy(eo_hbm.at[idx_v.at[j]], rows_v, sem).wait()
      pltpu.sync_copy(rows_v, out_hbm.at[pl.ds(base, CH)])

  return decode


def kernel(x, Wg, bg, W1, b1, W2, b2):
  orig_shape = x.shape
  M = x.shape[-1]
  xr = x.reshape(-1, M)
  S = xr.shape[0]
  E = Wg.shape[1]
  C = (S + E - 1) // E
  n_rows = E * C + 8

  K, CH = 2, 64
  assert S == _NW * K * CH

  slot, scale = _route(xr, Wg, bg, C, E, block_t=512)
  slot3 = slot.reshape(_NW, K, CH)
  scale3 = scale.reshape(_NW, K, CH)

  disp, gfs = _make_dispatch(S, M, n_rows, K, CH)(xr, slot3, scale3)
  eo = _ffn(disp, W1, b1, W2, b2, gfs[:E * C, None], C)
  rout = _make_decode(S, M, K, CH)(eo, slot3)
  return rout.reshape(orig_shape)


# speedup vs baseline: 1.0627x; 1.0627x over previous
import functools

import jax
import jax.numpy as jnp
from jax import lax
from jax.experimental import pallas as pl
from jax.experimental.pallas import tpu as pltpu
from jax.experimental.pallas import tpu_sc as plsc


def _route_body(C, E, x_ref, wg_ref, bg_ref, slot_ref, scale_ref, counts_ref):
  i = pl.program_id(0)

  @pl.when(i == 0)
  def _():
    counts_ref[...] = jnp.zeros_like(counts_ref)

  logits = jnp.dot(x_ref[...], wg_ref[...],
                   preferred_element_type=jnp.float32) + bg_ref[...]
  m = jnp.max(logits, axis=1, keepdims=True)
  gate = 1.0 / jnp.sum(jnp.exp(logits - m), axis=1, keepdims=True)
  e_iota = lax.broadcasted_iota(jnp.int32, logits.shape, 1)
  idx = jnp.min(jnp.where(logits == m, e_iota, E), axis=1,
                keepdims=True)
  onehot = (e_iota == idx).astype(jnp.float32)

  T = onehot.shape[0]
  r = lax.broadcasted_iota(jnp.int32, (T, T), 0)
  c = lax.broadcasted_iota(jnp.int32, (T, T), 1)
  tri = (c < r).astype(jnp.float32)
  loc = jnp.dot(tri, onehot, preferred_element_type=jnp.float32)
  loc = loc + counts_ref[...]
  counts_ref[...] = counts_ref[...] + jnp.sum(onehot, axis=0, keepdims=True)
  loc_i = jnp.sum(loc * onehot, axis=1, keepdims=True).astype(jnp.int32)

  valid = loc_i < C
  slot = idx * C + loc_i
  slot_ref[...] = jnp.where(valid, slot, E * C)
  scale_ref[...] = gate


def _route(xr, Wg, bg, C, E, block_t):
  S, M = xr.shape
  n = S // block_t
  return pl.pallas_call(
      functools.partial(_route_body, C, E),
      grid=(n,),
      in_specs=[
          pl.BlockSpec((block_t, M), lambda i: (i, 0)),
          pl.BlockSpec((M, E), lambda i: (0, 0)),
          pl.BlockSpec((1, E), lambda i: (0, 0)),
      ],
      out_specs=[
          pl.BlockSpec((block_t, 1), lambda i: (i, 0)),
          pl.BlockSpec((block_t, 1), lambda i: (i, 0)),
      ],
      out_shape=[
          jax.ShapeDtypeStruct((S, 1), jnp.int32),
          jax.ShapeDtypeStruct((S, 1), jnp.float32),
      ],
      scratch_shapes=[pltpu.VMEM((1, E), jnp.float32)],
  )(xr, Wg, bg.reshape(1, E))


def _ffn_body(E, x_ref, w1_ref, b1_ref, w2_ref, b2_ref, g_ref, out_ref):
  e = pl.program_id(0)
  f = pl.program_id(1)
  nf = pl.num_programs(1)

  @pl.when(e == E)
  def _():
    out_ref[...] = jnp.zeros_like(out_ref)

  @pl.when(e < E)
  def _():
    fb = w1_ref.shape[2]
    h = jnp.dot(x_ref[...], w1_ref[0], preferred_element_type=jnp.float32)
    b1blk = b1_ref[0, :, pl.ds(f * fb, fb)]
    h = jnp.maximum(h + b1blk, 0.0)
    contrib = jnp.dot(h, w2_ref[0], preferred_element_type=jnp.float32)

    @pl.when(f == 0)
    def _():
      out_ref[...] = contrib

    @pl.when(f != 0)
    def _():
      out_ref[...] += contrib

    @pl.when(f == nf - 1)
    def _():
      out_ref[...] = (out_ref[...] + b2_ref[0]) * g_ref[...]


def _ffn(disp, W1, b1, W2, b2, gfs, C):
  E, M, F = W1.shape
  nf = 2
  fb = F // nf
  ec = lambda e: jnp.minimum(e, E - 1)
  return pl.pallas_call(
      functools.partial(_ffn_body, E),
      grid=(E + 1, nf),
      in_specs=[
          pl.BlockSpec((C, M), lambda e, f: (ec(e), 0)),
          pl.BlockSpec((1, M, fb), lambda e, f: (ec(e), 0, f)),
          pl.BlockSpec((1, 1, F), lambda e, f: (ec(e), 0, 0)),
          pl.BlockSpec((1, fb, M), lambda e, f: (ec(e), f, 0)),
          pl.BlockSpec((1, 1, M), lambda e, f: (ec(e), 0, 0)),
          pl.BlockSpec((C, 1), lambda e, f: (ec(e), 0)),
      ],
      out_specs=pl.BlockSpec((C, M), lambda e, f: (e, 0)),
      out_shape=jax.ShapeDtypeStruct(((E + 1) * C, M), jnp.float32),
  )(disp, W1, b1.reshape(E, 1, F), W2, b2.reshape(E, 1, M), gfs)


_NC = 2
_NS = 16
_NW = _NC * _NS


def _make_dispatch(S, M, n_rows, K, CH):
  mesh = plsc.VectorSubcoreMesh(core_axis_name="c", subcore_axis_name="s")

  @functools.partial(
      pl.kernel,
      mesh=mesh,
      out_type=(
          jax.ShapeDtypeStruct((n_rows, M), jnp.float32),
          jax.ShapeDtypeStruct((n_rows,), jnp.float32),
      ),
      scratch_types=[
          pltpu.VMEM((K, CH), jnp.int32),
          pltpu.VMEM((CH, M), jnp.float32),
          pltpu.VMEM((CH, M), jnp.float32),
          pltpu.VMEM((K, CH), jnp.float32),
          pltpu.SemaphoreType.DMA,
          pltpu.SemaphoreType.DMA,
          pltpu.SemaphoreType.DMA,
          pltpu.SemaphoreType.DMA,
          pltpu.SemaphoreType.DMA,
      ],
  )
  def dispatch(x_hbm, slot_hbm, scale_hbm, out_hbm, gfs_hbm, idx_v, r0, r1,
               scale_v, l0, l1, s0, s1, gs):
    wid = lax.axis_index("s") * _NC + lax.axis_index("c")
    pltpu.sync_copy(slot_hbm.at[wid], idx_v)
    pltpu.sync_copy(scale_hbm.at[wid], scale_v)
    bufs, lsem, ssem = (r0, r1), (l0, l1), (s0, s1)
    scats = [None, None]
    gcopies = []
    for j in range(K):
      b = j % 2
      if scats[b] is not None:
        scats[b].wait()
      base = wid * (K * CH) + j * CH
      pltpu.async_copy(x_hbm.at[pl.ds(base, CH)], bufs[b], lsem[b]).wait()
      scats[b] = pltpu.async_copy(bufs[b], out_hbm.at[idx_v.at[j]], ssem[b])
      gcopies.append(
          pltpu.async_copy(scale_v.at[j], gfs_hbm.at[idx_v.at[j]], gs))
    for s in scats:
      s.wait()
    for g in gcopies:
      g.wait()

  return dispatch


def _make_decode(S, M, K, CH):
  mesh = plsc.VectorSubcoreMesh(core_axis_name="c", subcore_axis_name="s")

  @functools.partial(
      pl.kernel,
      mesh=mesh,
      out_type=jax.ShapeDtypeStruct((S, M), jnp.float32),
      scratch_types=[
          pltpu.VMEM((K, CH), jnp.int32),
          pltpu.VMEM((CH, M), jnp.float32),
          pltpu.VMEM((CH, M), jnp.float32),
          pltpu.SemaphoreType.DMA,
          pltpu.SemaphoreType.DMA,
          pltpu.SemaphoreType.DMA,
          pltpu.SemaphoreType.DMA,
      ],
  )
  def decode(eo_hbm, slot_hbm, out_hbm, idx_v, r0, r1, g0, g1, s0, s1):
    wid = lax.axis_index("s") * _NC + lax.axis_index("c")
    pltpu.sync_copy(slot_hbm.at[wid], idx_v)
    bufs, gsem, ssem = (r0, r1), (g0, g1), (s0, s1)
    stores = [None, None]
    for j in range(K):
      b = j % 2
      if stores[b] is not None:
        stores[b].wait()
      base = wid * (K * CH) + j * CH
      pltpu.async_copy(eo_hbm.at[idx_v.at[j]], bufs[b], gsem[b]).wait()
      stores[b] = pltpu.async_copy(bufs[b], out_hbm.at[pl.ds(base, CH)],
                                   ssem[b])
    for s in stores:
      s.wait()

  return decode


def kernel(x, Wg, bg, W1, b1, W2, b2):
  orig_shape = x.shape
  M = x.shape[-1]
  xr = x.reshape(-1, M)
  S = xr.shape[0]
  E = Wg.shape[1]
  C = (S + E - 1) // E
  n_rows = E * C + 8

  K, CH = 4, 32
  assert S == _NW * K * CH

  slot, scale = _route(xr, Wg, bg, C, E, block_t=512)
  slot3 = slot.reshape(_NW, K, CH)
  scale3 = scale.reshape(_NW, K, CH)

  disp, gfs = _make_dispatch(S, M, n_rows, K, CH)(xr, slot3, scale3)
  eo = _ffn(disp, W1, b1, W2, b2, gfs[:E * C, None], C)
  rout = _make_decode(S, M, K, CH)(eo, slot3)
  return rout.reshape(orig_shape)


# speedup vs baseline: 1.1185x; 1.0525x over previous
import functools

import jax
import jax.numpy as jnp
from jax import lax
from jax.experimental import pallas as pl
from jax.experimental.pallas import tpu as pltpu
from jax.experimental.pallas import tpu_sc as plsc


def _route_body(C, E, x_ref, wg_ref, bg_ref, slot_ref, scale_ref, counts_ref):
  i = pl.program_id(0)

  @pl.when(i == 0)
  def _():
    counts_ref[...] = jnp.zeros_like(counts_ref)

  logits = jnp.dot(x_ref[...], wg_ref[...],
                   preferred_element_type=jnp.float32) + bg_ref[...]
  m = jnp.max(logits, axis=1, keepdims=True)
  gate = 1.0 / jnp.sum(jnp.exp(logits - m), axis=1, keepdims=True)
  e_iota = lax.broadcasted_iota(jnp.int32, logits.shape, 1)
  idx = jnp.min(jnp.where(logits == m, e_iota, E), axis=1,
                keepdims=True)
  onehot = (e_iota == idx).astype(jnp.float32)

  T = onehot.shape[0]
  r = lax.broadcasted_iota(jnp.int32, (T, T), 0)
  c = lax.broadcasted_iota(jnp.int32, (T, T), 1)
  tri = (c < r).astype(jnp.float32)
  loc = jnp.dot(tri, onehot, preferred_element_type=jnp.float32)
  loc = loc + counts_ref[...]
  counts_ref[...] = counts_ref[...] + jnp.sum(onehot, axis=0, keepdims=True)
  loc_i = jnp.sum(loc * onehot, axis=1, keepdims=True).astype(jnp.int32)

  valid = loc_i < C
  slot = idx * C + loc_i
  slot_ref[...] = jnp.where(valid, slot, E * C)
  scale_ref[...] = gate


def _route(xr, Wg, bg, C, E, block_t):
  S, M = xr.shape
  n = S // block_t
  return pl.pallas_call(
      functools.partial(_route_body, C, E),
      grid=(n,),
      in_specs=[
          pl.BlockSpec((block_t, M), lambda i: (i, 0)),
          pl.BlockSpec((M, E), lambda i: (0, 0)),
          pl.BlockSpec((1, E), lambda i: (0, 0)),
      ],
      out_specs=[
          pl.BlockSpec((block_t, 1), lambda i: (i, 0)),
          pl.BlockSpec((block_t, 1), lambda i: (i, 0)),
      ],
      out_shape=[
          jax.ShapeDtypeStruct((S, 1), jnp.int32),
          jax.ShapeDtypeStruct((S, 1), jnp.float32),
      ],
      scratch_shapes=[pltpu.VMEM((1, E), jnp.float32)],
  )(xr, Wg, bg.reshape(1, E))


def _ffn_body(E, x_ref, w1_ref, b1_ref, w2_ref, b2_ref, g_ref, out_ref):
  e = pl.program_id(0)

  @pl.when(e == E)
  def _():
    out_ref[...] = jnp.zeros_like(out_ref)

  @pl.when(e < E)
  def _():
    h = jnp.dot(x_ref[...], w1_ref[0], preferred_element_type=jnp.float32)
    h = jnp.maximum(h + b1_ref[0], 0.0)
    contrib = jnp.dot(h, w2_ref[0], preferred_element_type=jnp.float32)
    out_ref[...] = (contrib + b2_ref[0]) * g_ref[...]


def _ffn(disp, W1, b1, W2, b2, gfs, C):
  E, M, F = W1.shape
  ec = lambda e: jnp.minimum(e, E - 1)
  return pl.pallas_call(
      functools.partial(_ffn_body, E),
      grid=(E + 1,),
      in_specs=[
          pl.BlockSpec((C, M), lambda e: (ec(e), 0)),
          pl.BlockSpec((1, M, F), lambda e: (ec(e), 0, 0)),
          pl.BlockSpec((1, 1, F), lambda e: (ec(e), 0, 0)),
          pl.BlockSpec((1, F, M), lambda e: (ec(e), 0, 0)),
          pl.BlockSpec((1, 1, M), lambda e: (ec(e), 0, 0)),
          pl.BlockSpec((C, 1), lambda e: (ec(e), 0)),
      ],
      out_specs=pl.BlockSpec((C, M), lambda e: (e, 0)),
      out_shape=jax.ShapeDtypeStruct(((E + 1) * C, M), jnp.float32),
  )(disp, W1, b1.reshape(E, 1, F), W2, b2.reshape(E, 1, M), gfs)


_NC = 2
_NS = 16
_NW = _NC * _NS


def _make_dispatch(S, M, n_rows, K, CH):
  mesh = plsc.VectorSubcoreMesh(core_axis_name="c", subcore_axis_name="s")

  @functools.partial(
      pl.kernel,
      mesh=mesh,
      out_type=(
          jax.ShapeDtypeStruct((n_rows, M), jnp.float32),
          jax.ShapeDtypeStruct((n_rows,), jnp.float32),
      ),
      scratch_types=[
          pltpu.VMEM((K, CH), jnp.int32),
          pltpu.VMEM((K * CH,), jnp.int32),
          pltpu.VMEM((K * CH,), jnp.float32),
          pltpu.VMEM((CH, M), jnp.float32),
          pltpu.VMEM((CH, M), jnp.float32),
          pltpu.SemaphoreType.DMA,
          pltpu.SemaphoreType.DMA,
          pltpu.SemaphoreType.DMA,
          pltpu.SemaphoreType.DMA,
          pltpu.SemaphoreType.DMA,
      ],
  )
  def dispatch(x_hbm, slot_hbm, scale_hbm, out_hbm, gfs_hbm, idx_v, idx_f,
               scale_f, r0, r1, l0, l1, s0, s1, gs):
    wid = lax.axis_index("s") * _NC + lax.axis_index("c")
    tbase = wid * (K * CH)
    pltpu.sync_copy(slot_hbm.at[pl.ds(tbase, K * CH)], idx_f)
    pltpu.sync_copy(scale_hbm.at[pl.ds(tbase, K * CH)], scale_f)
    for j in range(K):
      pltpu.sync_copy(slot_hbm.at[pl.ds(tbase + j * CH, CH)], idx_v.at[j])
    gcopy = pltpu.async_copy(scale_f, gfs_hbm.at[idx_f], gs)
    bufs, lsem, ssem = (r0, r1), (l0, l1), (s0, s1)
    scats = [None, None]
    for j in range(K):
      b = j % 2
      if scats[b] is not None:
        scats[b].wait()
      pltpu.async_copy(x_hbm.at[pl.ds(tbase + j * CH, CH)], bufs[b],
                       lsem[b]).wait()
      scats[b] = pltpu.async_copy(bufs[b], out_hbm.at[idx_v.at[j]], ssem[b])
    for s in scats:
      s.wait()
    gcopy.wait()

  return dispatch


def _make_decode(S, M, K, CH):
  mesh = plsc.VectorSubcoreMesh(core_axis_name="c", subcore_axis_name="s")

  @functools.partial(
      pl.kernel,
      mesh=mesh,
      out_type=jax.ShapeDtypeStruct((S, M), jnp.float32),
      scratch_types=[
          pltpu.VMEM((K, CH), jnp.int32),
          pltpu.VMEM((CH, M), jnp.float32),
          pltpu.VMEM((CH, M), jnp.float32),
          pltpu.SemaphoreType.DMA,
          pltpu.SemaphoreType.DMA,
          pltpu.SemaphoreType.DMA,
          pltpu.SemaphoreType.DMA,
      ],
  )
  def decode(eo_hbm, slot_hbm, out_hbm, idx_v, r0, r1, g0, g1, s0, s1):
    wid = lax.axis_index("s") * _NC + lax.axis_index("c")
    tbase = wid * (K * CH)
    for j in range(K):
      pltpu.sync_copy(slot_hbm.at[pl.ds(tbase + j * CH, CH)], idx_v.at[j])
    bufs, gsem, ssem = (r0, r1), (g0, g1), (s0, s1)
    stores = [None, None]
    for j in range(K):
      b = j % 2
      if stores[b] is not None:
        stores[b].wait()
      base = wid * (K * CH) + j * CH
      pltpu.async_copy(eo_hbm.at[idx_v.at[j]], bufs[b], gsem[b]).wait()
      stores[b] = pltpu.async_copy(bufs[b], out_hbm.at[pl.ds(base, CH)],
                                   ssem[b])
    for s in stores:
      s.wait()

  return decode


def kernel(x, Wg, bg, W1, b1, W2, b2):
  orig_shape = x.shape
  M = x.shape[-1]
  xr = x.reshape(-1, M)
  S = xr.shape[0]
  E = Wg.shape[1]
  C = (S + E - 1) // E
  n_rows = E * C + 8

  K, CH = 4, 32
  assert S == _NW * K * CH

  slot, scale = _route(xr, Wg, bg, C, E, block_t=1024)
  slot1 = slot.reshape(S)
  scale1 = scale.reshape(S)

  disp, gfs = _make_dispatch(S, M, n_rows, K, CH)(xr, slot1, scale1)
  eo = _ffn(disp, W1, b1, W2, b2, gfs[:E * C, None], C)
  rout = _make_decode(S, M, K, CH)(eo, slot1)
  return rout.reshape(orig_shape)
